# baseline (speedup 1.0000x reference)
import functools

import jax
import jax.numpy as jnp
from jax import lax
from jax.experimental import pallas as pl
from jax.experimental.pallas import tpu as pltpu
from jax.experimental.pallas import tpu_sc as plsc

N, K = 10000, 16
IN_DIM, H, MSG = 128, 256, 256
NP = 10240
NC, NS = 2, 16
NW = NC * NS
EDGES = NP * K
PER_W = EDGES // NW
CSZ = 128
CHUNKS = PER_W // CSZ
BN = 256
BNP = 512


def _pre_body(h_ref, feat_ref, w1_ref, b1_ref, wf_ref, bf_ref,
              q_ref, p_ref, f_ref):
    h = h_ref[...]
    q_ref[...] = jnp.dot(h, w1_ref[0:H, :],
                         preferred_element_type=jnp.float32) + b1_ref[...]
    p_ref[...] = jnp.dot(h, w1_ref[H:2 * H, :],
                         preferred_element_type=jnp.float32).astype(jnp.bfloat16)
    f_ref[...] = jnp.dot(feat_ref[...], wf_ref[...],
                         preferred_element_type=jnp.float32) + bf_ref[...]


def _precompute(hp, featp, W1, b1, W_feat, b_feat):
    grid = NP // BNP
    return pl.pallas_call(
        _pre_body,
        grid=(grid,),
        in_specs=[
            pl.BlockSpec((BNP, H), lambda i: (i, 0)),
            pl.BlockSpec((BNP, IN_DIM), lambda i: (i, 0)),
            pl.BlockSpec((2 * H, MSG), lambda i: (0, 0)),
            pl.BlockSpec((1, MSG), lambda i: (0, 0)),
            pl.BlockSpec((IN_DIM, H), lambda i: (0, 0)),
            pl.BlockSpec((1, H), lambda i: (0, 0)),
        ],
        out_specs=[
            pl.BlockSpec((BNP, MSG), lambda i: (i, 0)),
            pl.BlockSpec((BNP, MSG), lambda i: (i, 0)),
            pl.BlockSpec((BNP, H), lambda i: (i, 0)),
        ],
        out_shape=[
            jax.ShapeDtypeStruct((NP, MSG), jnp.float32),
            jax.ShapeDtypeStruct((NP, MSG), jnp.bfloat16),
            jax.ShapeDtypeStruct((NP, H), jnp.float32),
        ],
    )(hp, featp, W1, b1.reshape(1, MSG), W_feat, b_feat.reshape(1, H))


MW = MSG // 2


def _sc_gather(table, idx3):
    mesh = plsc.VectorSubcoreMesh(core_axis_name="c", subcore_axis_name="s")

    @functools.partial(
        pl.kernel, mesh=mesh,
        out_type=jax.ShapeDtypeStruct((EDGES, MW), jnp.int32),
        scratch_types=[
            pltpu.VMEM((CHUNKS, CSZ), jnp.int32),
            pltpu.VMEM((2, CSZ, MW), jnp.int32),
            pltpu.SemaphoreType.DMA,
            pltpu.SemaphoreType.DMA,
        ],
    )
    def k(table_hbm, idx_hbm, out_hbm, idx_v, rows_v, gsem, osem):
        wid = lax.axis_index("s") * NC + lax.axis_index("c")
        pltpu.sync_copy(idx_hbm.at[wid], idx_v)
        base = wid * PER_W

        pltpu.async_copy(table_hbm.at[idx_v.at[0]], rows_v.at[0], gsem)

        def body(j, carry):
            @pl.when(j >= 1)
            def _wait_out():
                pltpu.make_async_copy(
                    rows_v.at[(j - 1) % 2],
                    out_hbm.at[pl.ds(base + (j - 1) * CSZ, CSZ)],
                    osem).wait()

            @pl.when(j + 1 < CHUNKS)
            def _fire_next():
                pltpu.async_copy(table_hbm.at[idx_v.at[j + 1]],
                                 rows_v.at[(j + 1) % 2], gsem)

            pltpu.make_async_copy(table_hbm.at[idx_v.at[j]],
                                  rows_v.at[j % 2], gsem).wait()
            pltpu.async_copy(rows_v.at[j % 2],
                             out_hbm.at[pl.ds(base + j * CSZ, CSZ)], osem)
            return carry

        lax.fori_loop(0, CHUNKS, body, 0)
        pltpu.make_async_copy(
            rows_v.at[(CHUNKS - 1) % 2],
            out_hbm.at[pl.ds(base + (CHUNKS - 1) * CSZ, CSZ)],
            osem).wait()

    return k(table, idx3)


def _main_body(g_ref, q_ref, f_ref, h_ref, c_ref, w2_ref, b2_ref,
               wm_ref, wx_ref, wh_ref, bias_ref, ht_ref, ct_ref):
    q = q_ref[...]
    w2 = w2_ref[...]
    b2 = b2_ref[...]
    msg = jnp.zeros((BN, MSG), jnp.float32)
    for k in range(K):
        x = jnp.maximum(g_ref[k].astype(jnp.float32) + q, 0.0)
        y = jnp.dot(x, w2, preferred_element_type=jnp.float32) + b2
        msg = msg + jnp.maximum(y, 0.0)
    gates = (jnp.dot(msg, wm_ref[...], preferred_element_type=jnp.float32)
             + jnp.dot(f_ref[...], wx_ref[...], preferred_element_type=jnp.float32)
             + jnp.dot(h_ref[...], wh_ref[...], preferred_element_type=jnp.float32)
             + bias_ref[...])
    i_g = jax.nn.sigmoid(gates[:, 0 * H:1 * H])
    f_g = jax.nn.sigmoid(gates[:, 1 * H:2 * H])
    g_g = jnp.tanh(gates[:, 2 * H:3 * H])
    o_g = jax.nn.sigmoid(gates[:, 3 * H:4 * H])
    c_t = f_g * c_ref[...] + i_g * g_g
    ht_ref[...] = o_g * jnp.tanh(c_t)
    ct_ref[...] = c_t


def _main(G, Q, F, hp, cp, W2, b2, Wm, Wx, Wh, bias):
    grid = NP // BN
    return pl.pallas_call(
        _main_body,
        grid=(grid,),
        in_specs=[
            pl.BlockSpec((K, BN, MSG), lambda i: (0, i, 0)),

            pl.BlockSpec((BN, MSG), lambda i: (i, 0)),
            pl.BlockSpec((BN, H), lambda i: (i, 0)),
            pl.BlockSpec((BN, H), lambda i: (i, 0)),
            pl.BlockSpec((BN, H), lambda i: (i, 0)),
            pl.BlockSpec((MSG, MSG), lambda i: (0, 0)),
            pl.BlockSpec((1, MSG), lambda i: (0, 0)),
            pl.BlockSpec((MSG, 4 * H), lambda i: (0, 0)),
            pl.BlockSpec((H, 4 * H), lambda i: (0, 0)),
            pl.BlockSpec((H, 4 * H), lambda i: (0, 0)),
            pl.BlockSpec((1, 4 * H), lambda i: (0, 0)),
        ],
        out_specs=[
            pl.BlockSpec((BN, H), lambda i: (i, 0)),
            pl.BlockSpec((BN, H), lambda i: (i, 0)),
        ],
        out_shape=[
            jax.ShapeDtypeStruct((NP, H), jnp.float32),
            jax.ShapeDtypeStruct((NP, H), jnp.float32),
        ],
    )(G, Q, F, hp, cp, W2, b2, Wm, Wx, Wh, bias)


def kernel(feat_t, indices, h_t_minus_1, c_t_minus_1, W_feat, b_feat,
           W1, b1, W2, b2, W_ih, W_hh, b_ih, b_hh):
    h = h_t_minus_1[0]
    c = c_t_minus_1[0]
    feat = feat_t[0]
    idx = indices[0].astype(jnp.int32)

    pad = NP - N
    hp = jnp.pad(h, ((0, pad), (0, 0)))
    cp = jnp.pad(c, ((0, pad), (0, 0)))
    featp = jnp.pad(feat, ((0, pad), (0, 0)))
    idxp = jnp.pad(idx, ((0, pad), (0, 0)))

    Q, P, F = _precompute(hp, featp, W1, b1, W_feat, b_feat)

    idx_km = idxp.T.reshape(NW, CHUNKS, CSZ)
    P_i32 = jax.lax.bitcast_convert_type(
        P.reshape(NP, MW, 2), jnp.int32)
    G_i32 = _sc_gather(P_i32, idx_km)
    G = jax.lax.bitcast_convert_type(
        G_i32, jnp.bfloat16).reshape(K, NP, MSG)

    Wm = W_ih[:, :MSG].T
    Wx = W_ih[:, MSG:].T
    Wh = W_hh.T
    bias = (b_ih + b_hh).reshape(1, 4 * H)

    h_t, c_t = _main(G, Q, F, hp, cp, W2, b2.reshape(1, MSG), Wm, Wx, Wh, bias)
    return (h_t[:N][None], c_t[:N][None])


# speedup vs baseline: 3.4525x; 3.4525x over previous
import functools

import jax
import jax.numpy as jnp
from jax import lax
from jax.experimental import pallas as pl
from jax.experimental.pallas import tpu as pltpu
from jax.experimental.pallas import tpu_sc as plsc

N, K = 10000, 16
IN_DIM, H, MSG = 128, 256, 256
NP = 10240
NC, NS = 2, 16
NW = NC * NS
EDGES = NP * K
PER_W = EDGES // NW
CSZ = 128
CHUNKS = PER_W // CSZ
BN = 256
BNP = 512
MW = MSG // 2


def _rnd_bf16_bits(u):
    return (u + 0x7FFF + ((u >> 16) & 1)) >> 16


def _pre_body(h_ref, feat_ref, w1_ref, b1_ref, wf_ref, bf_ref,
              q_ref, p_ref, f_ref):
    h = h_ref[...]
    q_ref[...] = jnp.dot(h, w1_ref[0:H, :],
                         preferred_element_type=jnp.float32) + b1_ref[...]
    p = jnp.dot(h, w1_ref[H:2 * H, :], preferred_element_type=jnp.float32)
    ai = jax.lax.bitcast_convert_type(p[:, :H // 2], jnp.int32)
    bi = jax.lax.bitcast_convert_type(p[:, H // 2:], jnp.int32)
    p_ref[...] = ((_rnd_bf16_bits(ai) & 0xFFFF)
                  | (_rnd_bf16_bits(bi) << 16))
    f_ref[...] = jnp.dot(feat_ref[...], wf_ref[...],
                         preferred_element_type=jnp.float32) + bf_ref[...]


def _precompute(hp, featp, W1, b1, W_feat, b_feat):
    grid = NP // BNP
    return pl.pallas_call(
        _pre_body,
        grid=(grid,),
        in_specs=[
            pl.BlockSpec((BNP, H), lambda i: (i, 0)),
            pl.BlockSpec((BNP, IN_DIM), lambda i: (i, 0)),
            pl.BlockSpec((2 * H, MSG), lambda i: (0, 0)),
            pl.BlockSpec((1, MSG), lambda i: (0, 0)),
            pl.BlockSpec((IN_DIM, H), lambda i: (0, 0)),
            pl.BlockSpec((1, H), lambda i: (0, 0)),
        ],
        out_specs=[
            pl.BlockSpec((BNP, MSG), lambda i: (i, 0)),
            pl.BlockSpec((BNP, MW), lambda i: (i, 0)),
            pl.BlockSpec((BNP, H), lambda i: (i, 0)),
        ],
        out_shape=[
            jax.ShapeDtypeStruct((NP, MSG), jnp.float32),
            jax.ShapeDtypeStruct((NP, MW), jnp.int32),
            jax.ShapeDtypeStruct((NP, H), jnp.float32),
        ],
    )(hp, featp, W1, b1.reshape(1, MSG), W_feat, b_feat.reshape(1, H))


def _sc_gather(table, idx3):
    mesh = plsc.VectorSubcoreMesh(core_axis_name="c", subcore_axis_name="s")

    @functools.partial(
        pl.kernel, mesh=mesh,
        out_type=jax.ShapeDtypeStruct((EDGES, MW), jnp.int32),
        scratch_types=[
            pltpu.VMEM((CHUNKS, CSZ), jnp.int32),
            pltpu.VMEM((2, CSZ, MW), jnp.int32),
            pltpu.SemaphoreType.DMA,
            pltpu.SemaphoreType.DMA,
        ],
    )
    def k(table_hbm, idx_hbm, out_hbm, idx_v, rows_v, gsem, osem):
        wid = lax.axis_index("s") * NC + lax.axis_index("c")
        pltpu.sync_copy(idx_hbm.at[wid], idx_v)
        base = wid * PER_W

        pltpu.async_copy(table_hbm.at[idx_v.at[0]], rows_v.at[0], gsem)

        def body(j, carry):
            @pl.when(j >= 1)
            def _wait_out():
                pltpu.make_async_copy(
                    rows_v.at[(j - 1) % 2],
                    out_hbm.at[pl.ds(base + (j - 1) * CSZ, CSZ)],
                    osem).wait()

            @pl.when(j + 1 < CHUNKS)
            def _fire_next():
                pltpu.async_copy(table_hbm.at[idx_v.at[j + 1]],
                                 rows_v.at[(j + 1) % 2], gsem)

            pltpu.make_async_copy(table_hbm.at[idx_v.at[j]],
                                  rows_v.at[j % 2], gsem).wait()
            pltpu.async_copy(rows_v.at[j % 2],
                             out_hbm.at[pl.ds(base + j * CSZ, CSZ)], osem)
            return carry

        lax.fori_loop(0, CHUNKS, body, 0)
        pltpu.make_async_copy(
            rows_v.at[(CHUNKS - 1) % 2],
            out_hbm.at[pl.ds(base + (CHUNKS - 1) * CSZ, CSZ)],
            osem).wait()

    return k(table, idx3)


def _main_body(g_ref, q_ref, f_ref, h_ref, c_ref, w2_ref, b2_ref,
               wm_ref, wx_ref, wh_ref, bias_ref, ht_ref, ct_ref):
    q = q_ref[...]
    w2 = w2_ref[...]
    b2 = b2_ref[...]
    msg = jnp.zeros((BN, MSG), jnp.float32)
    for k in range(K):
        g = g_ref[k]
        lo = jax.lax.bitcast_convert_type(g << 16, jnp.float32)
        hi = jax.lax.bitcast_convert_type(g & jnp.int32(-65536), jnp.float32)
        x = jnp.maximum(jnp.concatenate([lo, hi], axis=1) + q, 0.0)
        y = jnp.dot(x, w2, preferred_element_type=jnp.float32) + b2
        msg = msg + jnp.maximum(y, 0.0)
    gates = (jnp.dot(msg, wm_ref[...], preferred_element_type=jnp.float32)
             + jnp.dot(f_ref[...], wx_ref[...], preferred_element_type=jnp.float32)
             + jnp.dot(h_ref[...], wh_ref[...], preferred_element_type=jnp.float32)
             + bias_ref[...])
    i_g = jax.nn.sigmoid(gates[:, 0 * H:1 * H])
    f_g = jax.nn.sigmoid(gates[:, 1 * H:2 * H])
    g_g = jnp.tanh(gates[:, 2 * H:3 * H])
    o_g = jax.nn.sigmoid(gates[:, 3 * H:4 * H])
    c_t = f_g * c_ref[...] + i_g * g_g
    ht_ref[...] = o_g * jnp.tanh(c_t)
    ct_ref[...] = c_t


def _main(G, Q, F, hp, cp, W2, b2, Wm, Wx, Wh, bias):
    grid = NP // BN
    return pl.pallas_call(
        _main_body,
        grid=(grid,),
        in_specs=[
            pl.BlockSpec((K, BN, MW), lambda i: (0, i, 0)),
            pl.BlockSpec((BN, MSG), lambda i: (i, 0)),
            pl.BlockSpec((BN, H), lambda i: (i, 0)),
            pl.BlockSpec((BN, H), lambda i: (i, 0)),
            pl.BlockSpec((BN, H), lambda i: (i, 0)),
            pl.BlockSpec((MSG, MSG), lambda i: (0, 0)),
            pl.BlockSpec((1, MSG), lambda i: (0, 0)),
            pl.BlockSpec((MSG, 4 * H), lambda i: (0, 0)),
            pl.BlockSpec((H, 4 * H), lambda i: (0, 0)),
            pl.BlockSpec((H, 4 * H), lambda i: (0, 0)),
            pl.BlockSpec((1, 4 * H), lambda i: (0, 0)),
        ],
        out_specs=[
            pl.BlockSpec((BN, H), lambda i: (i, 0)),
            pl.BlockSpec((BN, H), lambda i: (i, 0)),
        ],
        out_shape=[
            jax.ShapeDtypeStruct((NP, H), jnp.float32),
            jax.ShapeDtypeStruct((NP, H), jnp.float32),
        ],
    )(G, Q, F, hp, cp, W2, b2, Wm, Wx, Wh, bias)


def kernel(feat_t, indices, h_t_minus_1, c_t_minus_1, W_feat, b_feat,
           W1, b1, W2, b2, W_ih, W_hh, b_ih, b_hh):
    h = h_t_minus_1[0]
    c = c_t_minus_1[0]
    feat = feat_t[0]
    idx = indices[0].astype(jnp.int32)

    pad = NP - N
    hp = jnp.pad(h, ((0, pad), (0, 0)))
    cp = jnp.pad(c, ((0, pad), (0, 0)))
    featp = jnp.pad(feat, ((0, pad), (0, 0)))
    idxp = jnp.pad(idx, ((0, pad), (0, 0)))

    Q, P, F = _precompute(hp, featp, W1, b1, W_feat, b_feat)

    idx_km = idxp.T.reshape(NW, CHUNKS, CSZ)
    G = _sc_gather(P, idx_km).reshape(K, NP, MW)

    Wm = W_ih[:, :MSG].T
    Wx = W_ih[:, MSG:].T
    Wh = W_hh.T
    bias = (b_ih + b_hh).reshape(1, 4 * H)

    h_t, c_t = _main(G, Q, F, hp, cp, W2, b2.reshape(1, MSG), Wm, Wx, Wh, bias)
    return (h_t[:N][None], c_t[:N][None])


# speedup vs baseline: 3.4677x; 1.0044x over previous
import functools

import jax
import jax.numpy as jnp
from jax import lax
from jax.experimental import pallas as pl
from jax.experimental.pallas import tpu as pltpu
from jax.experimental.pallas import tpu_sc as plsc

N, K = 10000, 16
IN_DIM, H, MSG = 128, 256, 256
NP = 10240
NC, NS = 2, 16
NW = NC * NS
EDGES = NP * K
PER_W = EDGES // NW
CSZ = 128
CHUNKS = PER_W // CSZ
BN = 256
BNP = 512
MW = MSG // 2
NBUF = 4


def _rnd_bf16_bits(u):
    return (u + 0x7FFF + ((u >> 16) & 1)) >> 16


def _pre_body(h_ref, feat_ref, w1_ref, b1_ref, wf_ref, bf_ref,
              q_ref, p_ref, f_ref):
    h = h_ref[...]
    q_ref[...] = jnp.dot(h, w1_ref[0:H, :],
                         preferred_element_type=jnp.float32) + b1_ref[...]
    p = jnp.dot(h, w1_ref[H:2 * H, :], preferred_element_type=jnp.float32)
    ai = jax.lax.bitcast_convert_type(p[:, :H // 2], jnp.int32)
    bi = jax.lax.bitcast_convert_type(p[:, H // 2:], jnp.int32)
    p_ref[...] = ((_rnd_bf16_bits(ai) & 0xFFFF)
                  | (_rnd_bf16_bits(bi) << 16))
    f_ref[...] = jnp.dot(feat_ref[...], wf_ref[...],
                         preferred_element_type=jnp.float32) + bf_ref[...]


def _precompute(hp, featp, W1, b1, W_feat, b_feat):
    grid = NP // BNP
    return pl.pallas_call(
        _pre_body,
        grid=(grid,),
        in_specs=[
            pl.BlockSpec((BNP, H), lambda i: (i, 0)),
            pl.BlockSpec((BNP, IN_DIM), lambda i: (i, 0)),
            pl.BlockSpec((2 * H, MSG), lambda i: (0, 0)),
            pl.BlockSpec((1, MSG), lambda i: (0, 0)),
            pl.BlockSpec((IN_DIM, H), lambda i: (0, 0)),
            pl.BlockSpec((1, H), lambda i: (0, 0)),
        ],
        out_specs=[
            pl.BlockSpec((BNP, MSG), lambda i: (i, 0)),
            pl.BlockSpec((BNP, MW), lambda i: (i, 0)),
            pl.BlockSpec((BNP, H), lambda i: (i, 0)),
        ],
        out_shape=[
            jax.ShapeDtypeStruct((NP, MSG), jnp.float32),
            jax.ShapeDtypeStruct((NP, MW), jnp.int32),
            jax.ShapeDtypeStruct((NP, H), jnp.float32),
        ],
    )(hp, featp, W1, b1.reshape(1, MSG), W_feat, b_feat.reshape(1, H))


def _sc_gather(table, idx3):
    mesh = plsc.VectorSubcoreMesh(core_axis_name="c", subcore_axis_name="s")

    @functools.partial(
        pl.kernel, mesh=mesh,
        out_type=jax.ShapeDtypeStruct((EDGES, MW), jnp.int32),
        scratch_types=[
            pltpu.VMEM((CHUNKS, CSZ), jnp.int32),
            pltpu.VMEM((NBUF, CSZ, MW), jnp.int32),
            pltpu.SemaphoreType.DMA,
            pltpu.SemaphoreType.DMA,
        ],
    )
    def k(table_hbm, idx_hbm, out_hbm, idx_v, rows_v, gsem, osem):
        wid = lax.axis_index("s") * NC + lax.axis_index("c")
        pltpu.sync_copy(idx_hbm.at[wid], idx_v)
        base = wid * PER_W

        for b in range(NBUF - 1):
            pltpu.async_copy(table_hbm.at[idx_v.at[b]], rows_v.at[b], gsem)

        def body(j, carry):
            @pl.when(j >= 1)
            def _wait_out():
                pltpu.make_async_copy(
                    rows_v.at[(j - 1) % NBUF],
                    out_hbm.at[pl.ds(base + (j - 1) * CSZ, CSZ)],
                    osem).wait()

            @pl.when(j + NBUF - 1 < CHUNKS)
            def _fire_next():
                pltpu.async_copy(table_hbm.at[idx_v.at[j + NBUF - 1]],
                                 rows_v.at[(j - 1) % NBUF], gsem)

            pltpu.make_async_copy(table_hbm.at[idx_v.at[j]],
                                  rows_v.at[j % NBUF], gsem).wait()
            pltpu.async_copy(rows_v.at[j % NBUF],
                             out_hbm.at[pl.ds(base + j * CSZ, CSZ)], osem)
            return carry

        lax.fori_loop(0, CHUNKS, body, 0)
        pltpu.make_async_copy(
            rows_v.at[(CHUNKS - 1) % NBUF],
            out_hbm.at[pl.ds(base + (CHUNKS - 1) * CSZ, CSZ)],
            osem).wait()

    return k(table, idx3)


def _main_body(g_ref, q_ref, f_ref, h_ref, c_ref, w2_ref, b2_ref,
               wm_ref, wx_ref, wh_ref, bias_ref, ht_ref, ct_ref):
    q = q_ref[...]
    w2 = w2_ref[...]
    b2 = b2_ref[...]
    msg = jnp.zeros((BN, MSG), jnp.float32)
    for k in range(K):
        g = g_ref[k]
        lo = jax.lax.bitcast_convert_type(g << 16, jnp.float32)
        hi = jax.lax.bitcast_convert_type(g & jnp.int32(-65536), jnp.float32)
        x = jnp.maximum(jnp.concatenate([lo, hi], axis=1) + q, 0.0)
        y = jnp.dot(x, w2, preferred_element_type=jnp.float32) + b2
        msg = msg + jnp.maximum(y, 0.0)
    gates = (jnp.dot(msg, wm_ref[...], preferred_element_type=jnp.float32)
             + jnp.dot(f_ref[...], wx_ref[...], preferred_element_type=jnp.float32)
             + jnp.dot(h_ref[...], wh_ref[...], preferred_element_type=jnp.float32)
             + bias_ref[...])
    i_g = jax.nn.sigmoid(gates[:, 0 * H:1 * H])
    f_g = jax.nn.sigmoid(gates[:, 1 * H:2 * H])
    g_g = jnp.tanh(gates[:, 2 * H:3 * H])
    o_g = jax.nn.sigmoid(gates[:, 3 * H:4 * H])
    c_t = f_g * c_ref[...] + i_g * g_g
    ht_ref[...] = o_g * jnp.tanh(c_t)
    ct_ref[...] = c_t


def _main(G, Q, F, hp, cp, W2, b2, Wm, Wx, Wh, bias):
    grid = NP // BN
    return pl.pallas_call(
        _main_body,
        grid=(grid,),
        in_specs=[
            pl.BlockSpec((K, BN, MW), lambda i: (0, i, 0)),
            pl.BlockSpec((BN, MSG), lambda i: (i, 0)),
            pl.BlockSpec((BN, H), lambda i: (i, 0)),
            pl.BlockSpec((BN, H), lambda i: (i, 0)),
            pl.BlockSpec((BN, H), lambda i: (i, 0)),
            pl.BlockSpec((MSG, MSG), lambda i: (0, 0)),
            pl.BlockSpec((1, MSG), lambda i: (0, 0)),
            pl.BlockSpec((MSG, 4 * H), lambda i: (0, 0)),
            pl.BlockSpec((H, 4 * H), lambda i: (0, 0)),
            pl.BlockSpec((H, 4 * H), lambda i: (0, 0)),
            pl.BlockSpec((1, 4 * H), lambda i: (0, 0)),
        ],
        out_specs=[
            pl.BlockSpec((BN, H), lambda i: (i, 0)),
            pl.BlockSpec((BN, H), lambda i: (i, 0)),
        ],
        out_shape=[
            jax.ShapeDtypeStruct((NP, H), jnp.float32),
            jax.ShapeDtypeStruct((NP, H), jnp.float32),
        ],
    )(G, Q, F, hp, cp, W2, b2, Wm, Wx, Wh, bias)


def kernel(feat_t, indices, h_t_minus_1, c_t_minus_1, W_feat, b_feat,
           W1, b1, W2, b2, W_ih, W_hh, b_ih, b_hh):
    h = h_t_minus_1[0]
    c = c_t_minus_1[0]
    feat = feat_t[0]
    idx = indices[0].astype(jnp.int32)

    pad = NP - N
    hp = jnp.pad(h, ((0, pad), (0, 0)))
    cp = jnp.pad(c, ((0, pad), (0, 0)))
    featp = jnp.pad(feat, ((0, pad), (0, 0)))
    idxp = jnp.pad(idx, ((0, pad), (0, 0)))

    Q, P, F = _precompute(hp, featp, W1, b1, W_feat, b_feat)

    idx_km = idxp.T.reshape(NW, CHUNKS, CSZ)
    G = _sc_gather(P, idx_km).reshape(K, NP, MW)

    Wm = W_ih[:, :MSG].T
    Wx = W_ih[:, MSG:].T
    Wh = W_hh.T
    bias = (b_ih + b_hh).reshape(1, 4 * H)

    h_t, c_t = _main(G, Q, F, hp, cp, W2, b2.reshape(1, MSG), Wm, Wx, Wh, bias)
    return (h_t[:N][None], c_t[:N][None])
